# Initial kernel scaffold; baseline (speedup 1.0000x reference)
import jax
import jax.numpy as jnp
from jax.experimental import pallas as pl


def kernel(x, edge_index, W_mu, b_mu, W_logstd, b_logstd):
    raise NotImplementedError("write your pallas kernel here")


# speedup vs baseline: 20.4099x; 20.4099x over previous
import functools

import jax
import jax.numpy as jnp
from jax import lax
from jax.experimental import pallas as pl
from jax.experimental.pallas import tpu as pltpu
from jax.experimental.pallas import tpu_sc as plsc

N_NODES = 10000
D = 128
N_EDGES = 320000

NC = 2
NS = 16
L = 16
NW = NC * NS

CHUNK = 64
GC = 16
NG = 10
NCH = GC * NG
EPW = NCH * CHUNK
E_PAD = EPW * NW
N_PAD = 10240
RPT = N_PAD // NS

_mesh = plsc.VectorSubcoreMesh(core_axis_name="c", subcore_axis_name="s")


@functools.partial(
    pl.kernel,
    out_type=jax.ShapeDtypeStruct((NC, N_PAD), jnp.float32),
    mesh=_mesh,
    scratch_types=[
        pltpu.VMEM((GC, CHUNK), jnp.int32),
        pltpu.VMEM((CHUNK,), jnp.float32),
        pltpu.VMEM((RPT,), jnp.float32),
        pltpu.VMEM_SHARED((N_PAD,), jnp.float32),
        pltpu.SemaphoreType.DMA,
    ],
)
def _deg_kernel(dst_hbm, deg_out, idx_v, ones_v, zero_v, deg_sh, sem):
    cid = lax.axis_index("c")
    sid = lax.axis_index("s")
    wid = cid * NS + sid

    zeros = jnp.zeros((L,), jnp.float32)
    ones = jnp.ones((L,), jnp.float32)

    def _zero(i, carry):
        zero_v[pl.ds(i * L, L)] = zeros
        return carry

    lax.fori_loop(0, RPT // L, _zero, 0)

    def _one(i, carry):
        ones_v[pl.ds(i * L, L)] = ones
        return carry

    lax.fori_loop(0, CHUNK // L, _one, 0)

    pltpu.sync_copy(zero_v, deg_sh.at[pl.ds(sid * RPT, RPT)])
    plsc.subcore_barrier()

    def _group(g, carry):
        pltpu.sync_copy(dst_hbm.at[wid, g], idx_v)

        def _scat(k, c2):
            pltpu.async_copy(ones_v, deg_sh.at[idx_v.at[k]], sem, add=True)
            return c2

        lax.fori_loop(0, GC, _scat, 0)

        def _drain(k, c2):
            pltpu.make_async_copy(ones_v, deg_sh.at[idx_v.at[k]], sem).wait()
            return c2

        lax.fori_loop(0, GC, _drain, 0)
        return carry

    lax.fori_loop(0, NG, _group, 0)

    plsc.subcore_barrier()
    pltpu.sync_copy(deg_sh.at[pl.ds(sid * RPT, RPT)],
                    deg_out.at[cid, pl.ds(sid * RPT, RPT)])


@functools.partial(
    pl.kernel,
    out_type=jax.ShapeDtypeStruct((NC, N_PAD, D), jnp.float32),
    mesh=_mesh,
    scratch_types=[
        pltpu.VMEM((GC, CHUNK), jnp.int32),
        pltpu.VMEM((GC, CHUNK), jnp.int32),
        pltpu.VMEM((GC, CHUNK), jnp.int32),
        pltpu.VMEM((GC, CHUNK), jnp.int32),
        pltpu.VMEM((CHUNK, D), jnp.float32),
        pltpu.VMEM((CHUNK, D), jnp.float32),
        pltpu.VMEM_SHARED((N_PAD, D), jnp.float32),
        pltpu.SemaphoreType.DMA,
        pltpu.SemaphoreType.DMA,
        pltpu.SemaphoreType.DMA,
    ],
)
def _agg_kernel(xs_hbm, src_hbm, dst_hbm, part_out,
                idxs0, idxs1, idxd0, idxd1, rows0, rows1, agg_sh,
                sem0, sem1, semi):
    cid = lax.axis_index("c")
    sid = lax.axis_index("s")
    wid = cid * NS + sid

    zeros = jnp.zeros((L,), jnp.float32)

    def _zero(t, carry):
        r = t // (D // L)
        j = t - r * (D // L)
        rows0[r, pl.ds(j * L, L)] = zeros
        return carry

    lax.fori_loop(0, CHUNK * (D // L), _zero, 0)

    for q in range(RPT // CHUNK):
        pltpu.sync_copy(rows0, agg_sh.at[pl.ds(sid * RPT + q * CHUNK, CHUNK)])
    plsc.subcore_barrier()

    pltpu.sync_copy(src_hbm.at[wid, 0], idxs0)
    pltpu.sync_copy(dst_hbm.at[wid, 0], idxd0)
    pltpu.async_copy(src_hbm.at[wid, 1], idxs1, semi)
    pltpu.async_copy(dst_hbm.at[wid, 1], idxd1, semi)
    pltpu.async_copy(xs_hbm.at[idxs0.at[0]], rows0, sem0)

    for g in range(NG):
        a_idxs, a_idxd = (idxs0, idxd0) if g % 2 == 0 else (idxs1, idxd1)
        n_idxs, n_idxd = (idxs1, idxd1) if g % 2 == 0 else (idxs0, idxd0)

        def _step(i, carry, a_idxs=a_idxs, a_idxd=a_idxd):
            k0 = i * 2
            k1 = k0 + 1
            pltpu.async_copy(xs_hbm.at[a_idxs.at[k1]], rows1, sem1)
            pltpu.make_async_copy(xs_hbm.at[a_idxs.at[k0]], rows0, sem0).wait()
            pltpu.sync_copy(rows0, agg_sh.at[a_idxd.at[k0]], add=True)
            pltpu.async_copy(xs_hbm.at[a_idxs.at[k0 + 2]], rows0, sem0)
            pltpu.make_async_copy(xs_hbm.at[a_idxs.at[k1]], rows1, sem1).wait()
            pltpu.sync_copy(rows1, agg_sh.at[a_idxd.at[k1]], add=True)
            return carry

        lax.fori_loop(0, GC // 2 - 1, _step, 0)

        pltpu.async_copy(xs_hbm.at[a_idxs.at[GC - 1]], rows1, sem1)
        pltpu.make_async_copy(xs_hbm.at[a_idxs.at[GC - 2]], rows0, sem0).wait()
        pltpu.sync_copy(rows0, agg_sh.at[a_idxd.at[GC - 2]], add=True)
        if g + 1 < NG:
            pltpu.make_async_copy(src_hbm.at[wid, g + 1], n_idxs, semi).wait()
            pltpu.make_async_copy(dst_hbm.at[wid, g + 1], n_idxd, semi).wait()
            pltpu.async_copy(xs_hbm.at[n_idxs.at[0]], rows0, sem0)
        pltpu.make_async_copy(xs_hbm.at[a_idxs.at[GC - 1]], rows1, sem1).wait()
        pltpu.sync_copy(rows1, agg_sh.at[a_idxd.at[GC - 1]], add=True)
        if g + 2 < NG:
            pltpu.async_copy(src_hbm.at[wid, g + 2], a_idxs, semi)
            pltpu.async_copy(dst_hbm.at[wid, g + 2], a_idxd, semi)

    plsc.subcore_barrier()
    pltpu.sync_copy(agg_sh.at[pl.ds(sid * RPT, RPT)],
                    part_out.at[cid, pl.ds(sid * RPT, RPT)])


def _scale_body(x_ref, degp_ref, xs_ref):
    dis = lax.rsqrt(degp_ref[0] + degp_ref[1] + 1.0)
    xs_ref[...] = x_ref[...] * dis


def _scale_call(x_pad, degp3):
    blk = N_PAD // 8
    return pl.pallas_call(
        _scale_body,
        grid=(8,),
        in_specs=[
            pl.BlockSpec((blk, D), lambda i: (i, 0)),
            pl.BlockSpec((NC, blk, 1), lambda i: (0, i, 0)),
        ],
        out_specs=pl.BlockSpec((blk, D), lambda i: (i, 0)),
        out_shape=jax.ShapeDtypeStruct((N_PAD, D), jnp.float32),
    )(x_pad, degp3)


def _out_body(p_ref, xs_ref, degp_ref, wmu_ref, bmu_ref, wls_ref, bls_ref,
              mu_ref, ls_ref):
    dis = lax.rsqrt(degp_ref[0] + degp_ref[1] + 1.0)
    y = (p_ref[0] + p_ref[1] + xs_ref[...]) * dis
    mu_ref[...] = (
        jnp.dot(y, wmu_ref[...], preferred_element_type=jnp.float32)
        + bmu_ref[...]
    )
    ls_ref[...] = (
        jnp.dot(y, wls_ref[...], preferred_element_type=jnp.float32)
        + bls_ref[...]
    )


def _out_call(part, xs, degp3, W_mu, b_mu, W_logstd, b_logstd):
    blk = 1000
    grid = N_NODES // blk
    wspec = pl.BlockSpec((D, D), lambda i: (0, 0))
    bspec = pl.BlockSpec((1, D), lambda i: (0, 0))
    return pl.pallas_call(
        _out_body,
        grid=(grid,),
        in_specs=[
            pl.BlockSpec((NC, blk, D), lambda i: (0, i, 0)),
            pl.BlockSpec((blk, D), lambda i: (i, 0)),
            pl.BlockSpec((NC, blk, 1), lambda i: (0, i, 0)),
            wspec, bspec, wspec, bspec,
        ],
        out_specs=[
            pl.BlockSpec((blk, D), lambda i: (i, 0)),
            pl.BlockSpec((blk, D), lambda i: (i, 0)),
        ],
        out_shape=[
            jax.ShapeDtypeStruct((N_NODES, D), jnp.float32),
            jax.ShapeDtypeStruct((N_NODES, D), jnp.float32),
        ],
    )(part, xs, degp3, W_mu, b_mu, W_logstd, b_logstd)


def kernel(x, edge_index, W_mu, b_mu, W_logstd, b_logstd):
    src = edge_index[0].astype(jnp.int32)
    dst = edge_index[1].astype(jnp.int32)
    pad = jnp.full((E_PAD - N_EDGES,), N_NODES, jnp.int32)
    srcb = jnp.concatenate([src, pad]).reshape(NW, NG, GC, CHUNK)
    dstb = jnp.concatenate([dst, pad]).reshape(NW, NG, GC, CHUNK)
    x_pad = jnp.zeros((N_PAD, D), jnp.float32).at[:N_NODES].set(x)

    degp = _deg_kernel(dstb)
    degp3 = degp.reshape(NC, N_PAD, 1)
    xs = _scale_call(x_pad, degp3)
    part = _agg_kernel(xs, srcb, dstb)
    mu, logstd = _out_call(part, xs, degp3, W_mu, b_mu.reshape(1, D),
                           W_logstd, b_logstd.reshape(1, D))
    return (mu, logstd)


# speedup vs baseline: 47.0786x; 2.3067x over previous
import functools

import jax
import jax.numpy as jnp
from jax import lax
from jax.experimental import pallas as pl
from jax.experimental.pallas import tpu as pltpu
from jax.experimental.pallas import tpu_sc as plsc

N_NODES = 10000
D = 128
N_EDGES = 320000

NC = 2
NS = 16
L = 16
NW = NC * NS

CHUNK = 64
GC = 16
NG = 10
NCH = GC * NG
EPW = NCH * CHUNK
E_PAD = EPW * NW
N_PAD = 10240
RPT = N_PAD // NS

_mesh = plsc.VectorSubcoreMesh(core_axis_name="c", subcore_axis_name="s")


@functools.partial(
    pl.kernel,
    out_type=jax.ShapeDtypeStruct((NC, N_PAD), jnp.float32),
    mesh=_mesh,
    scratch_types=[
        pltpu.VMEM((GC, CHUNK), jnp.int32),
        pltpu.VMEM((CHUNK,), jnp.float32),
        pltpu.VMEM((RPT,), jnp.float32),
        pltpu.VMEM_SHARED((N_PAD,), jnp.float32),
        pltpu.SemaphoreType.DMA,
    ],
)
def _deg_kernel(dst_hbm, deg_out, idx_v, ones_v, zero_v, deg_sh, sem):
    cid = lax.axis_index("c")
    sid = lax.axis_index("s")
    wid = cid * NS + sid

    zeros = jnp.zeros((L,), jnp.float32)
    ones = jnp.ones((L,), jnp.float32)

    def _zero(i, carry):
        zero_v[pl.ds(i * L, L)] = zeros
        return carry

    lax.fori_loop(0, RPT // L, _zero, 0)

    def _one(i, carry):
        ones_v[pl.ds(i * L, L)] = ones
        return carry

    lax.fori_loop(0, CHUNK // L, _one, 0)

    pltpu.sync_copy(zero_v, deg_sh.at[pl.ds(sid * RPT, RPT)])
    plsc.subcore_barrier()

    def _group(g, carry):
        pltpu.sync_copy(dst_hbm.at[wid, g], idx_v)

        def _scat(k, c2):
            pltpu.async_copy(ones_v, deg_sh.at[idx_v.at[k]], sem, add=True)
            return c2

        lax.fori_loop(0, GC, _scat, 0)

        def _drain(k, c2):
            pltpu.make_async_copy(ones_v, deg_sh.at[idx_v.at[k]], sem).wait()
            return c2

        lax.fori_loop(0, GC, _drain, 0)
        return carry

    lax.fori_loop(0, NG, _group, 0)

    plsc.subcore_barrier()
    pltpu.sync_copy(deg_sh.at[pl.ds(sid * RPT, RPT)],
                    deg_out.at[cid, pl.ds(sid * RPT, RPT)])


@functools.partial(
    pl.kernel,
    out_type=jax.ShapeDtypeStruct((NC, N_PAD, D), jnp.float32),
    mesh=_mesh,
    scratch_types=[
        pltpu.VMEM((GC, CHUNK), jnp.int32),
        pltpu.VMEM((GC, CHUNK), jnp.int32),
        pltpu.VMEM((GC, CHUNK), jnp.int32),
        pltpu.VMEM((GC, CHUNK), jnp.int32),
        pltpu.VMEM((CHUNK, D), jnp.float32),
        pltpu.VMEM((CHUNK, D), jnp.float32),
        pltpu.VMEM_SHARED((N_PAD, D), jnp.float32),
        pltpu.SemaphoreType.DMA,
        pltpu.SemaphoreType.DMA,
        pltpu.SemaphoreType.DMA,
    ],
)
def _agg_kernel(xs_hbm, src_hbm, dst_hbm, part_out,
                idxs0, idxs1, idxd0, idxd1, rows0, rows1, agg_sh,
                sem0, sem1, semi):
    cid = lax.axis_index("c")
    sid = lax.axis_index("s")
    wid = cid * NS + sid

    zeros = jnp.zeros((L,), jnp.float32)

    def _zero(t, carry):
        r = t // (D // L)
        j = t - r * (D // L)
        rows0[r, pl.ds(j * L, L)] = zeros
        return carry

    lax.fori_loop(0, CHUNK * (D // L), _zero, 0)

    for q in range(RPT // CHUNK):
        pltpu.sync_copy(rows0, agg_sh.at[pl.ds(sid * RPT + q * CHUNK, CHUNK)])
    plsc.subcore_barrier()

    pltpu.sync_copy(src_hbm.at[wid, 0], idxs0)
    pltpu.sync_copy(dst_hbm.at[wid, 0], idxd0)
    pltpu.async_copy(src_hbm.at[wid, 1], idxs1, semi)
    pltpu.async_copy(dst_hbm.at[wid, 1], idxd1, semi)
    pltpu.async_copy(xs_hbm.at[idxs0.at[0]], rows0, sem0)

    for g in range(NG):
        a_idxs, a_idxd = (idxs0, idxd0) if g % 2 == 0 else (idxs1, idxd1)
        n_idxs, n_idxd = (idxs1, idxd1) if g % 2 == 0 else (idxs0, idxd0)

        def _step(i, carry, a_idxs=a_idxs, a_idxd=a_idxd):
            k0 = i * 2
            k1 = k0 + 1
            pltpu.async_copy(xs_hbm.at[a_idxs.at[k1]], rows1, sem1)
            pltpu.make_async_copy(xs_hbm.at[a_idxs.at[k0]], rows0, sem0).wait()
            pltpu.sync_copy(rows0, agg_sh.at[a_idxd.at[k0]], add=True)
            pltpu.async_copy(xs_hbm.at[a_idxs.at[k0 + 2]], rows0, sem0)
            pltpu.make_async_copy(xs_hbm.at[a_idxs.at[k1]], rows1, sem1).wait()
            pltpu.sync_copy(rows1, agg_sh.at[a_idxd.at[k1]], add=True)
            return carry

        lax.fori_loop(0, GC // 2 - 1, _step, 0)

        pltpu.async_copy(xs_hbm.at[a_idxs.at[GC - 1]], rows1, sem1)
        pltpu.make_async_copy(xs_hbm.at[a_idxs.at[GC - 2]], rows0, sem0).wait()
        pltpu.sync_copy(rows0, agg_sh.at[a_idxd.at[GC - 2]], add=True)
        if g + 1 < NG:
            pltpu.make_async_copy(src_hbm.at[wid, g + 1], n_idxs, semi).wait()
            pltpu.make_async_copy(dst_hbm.at[wid, g + 1], n_idxd, semi).wait()
            pltpu.async_copy(xs_hbm.at[n_idxs.at[0]], rows0, sem0)
        pltpu.make_async_copy(xs_hbm.at[a_idxs.at[GC - 1]], rows1, sem1).wait()
        pltpu.sync_copy(rows1, agg_sh.at[a_idxd.at[GC - 1]], add=True)
        if g + 2 < NG:
            pltpu.async_copy(src_hbm.at[wid, g + 2], a_idxs, semi)
            pltpu.async_copy(dst_hbm.at[wid, g + 2], a_idxd, semi)

    plsc.subcore_barrier()
    pltpu.sync_copy(agg_sh.at[pl.ds(sid * RPT, RPT)],
                    part_out.at[cid, pl.ds(sid * RPT, RPT)])


def _scale_body(x_ref, degp_ref, xs_ref):
    dis = lax.rsqrt(degp_ref[0] + degp_ref[1] + 1.0)
    xs_ref[...] = x_ref[...] * dis


def _scale_call(x_pad, degp3):
    blk = N_PAD // 8
    return pl.pallas_call(
        _scale_body,
        grid=(8,),
        in_specs=[
            pl.BlockSpec((blk, D), lambda i: (i, 0)),
            pl.BlockSpec((NC, blk, 1), lambda i: (0, i, 0)),
        ],
        out_specs=pl.BlockSpec((blk, D), lambda i: (i, 0)),
        out_shape=jax.ShapeDtypeStruct((N_PAD, D), jnp.float32),
    )(x_pad, degp3)


def _out_body(p_ref, xs_ref, degp_ref, wmu_ref, bmu_ref, wls_ref, bls_ref,
              mu_ref, ls_ref):
    dis = lax.rsqrt(degp_ref[0] + degp_ref[1] + 1.0)
    y = (p_ref[0] + p_ref[1] + xs_ref[...]) * dis
    mu_ref[...] = (
        jnp.dot(y, wmu_ref[...], preferred_element_type=jnp.float32)
        + bmu_ref[...]
    )
    ls_ref[...] = (
        jnp.dot(y, wls_ref[...], preferred_element_type=jnp.float32)
        + bls_ref[...]
    )


def _out_call(part, xs, degp3, W_mu, b_mu, W_logstd, b_logstd):
    blk = 1000
    grid = N_NODES // blk
    wspec = pl.BlockSpec((D, D), lambda i: (0, 0))
    bspec = pl.BlockSpec((1, D), lambda i: (0, 0))
    return pl.pallas_call(
        _out_body,
        grid=(grid,),
        in_specs=[
            pl.BlockSpec((NC, blk, D), lambda i: (0, i, 0)),
            pl.BlockSpec((blk, D), lambda i: (i, 0)),
            pl.BlockSpec((NC, blk, 1), lambda i: (0, i, 0)),
            wspec, bspec, wspec, bspec,
        ],
        out_specs=[
            pl.BlockSpec((blk, D), lambda i: (i, 0)),
            pl.BlockSpec((blk, D), lambda i: (i, 0)),
        ],
        out_shape=[
            jax.ShapeDtypeStruct((N_NODES, D), jnp.float32),
            jax.ShapeDtypeStruct((N_NODES, D), jnp.float32),
        ],
    )(part, xs, degp3, W_mu, b_mu, W_logstd, b_logstd)


def kernel(x, edge_index, W_mu, b_mu, W_logstd, b_logstd):
    src = edge_index[0].astype(jnp.int32)
    dst = edge_index[1].astype(jnp.int32)
    pad = N_NODES + (jnp.arange(E_PAD - N_EDGES, dtype=jnp.int32)
                     % (N_PAD - N_NODES))
    srcb = jnp.concatenate([src, pad]).reshape(NW, NG, GC, CHUNK)
    dstb = jnp.concatenate([dst, pad]).reshape(NW, NG, GC, CHUNK)
    x_pad = jnp.zeros((N_PAD, D), jnp.float32).at[:N_NODES].set(x)

    degp = _deg_kernel(dstb)
    degp3 = degp.reshape(NC, N_PAD, 1)
    xs = _scale_call(x_pad, degp3)
    part = _agg_kernel(xs, srcb, dstb)
    mu, logstd = _out_call(part, xs, degp3, W_mu, b_mu.reshape(1, D),
                           W_logstd, b_logstd.reshape(1, D))
    return (mu, logstd)


# speedup vs baseline: 53.0651x; 1.1272x over previous
import functools

import jax
import jax.numpy as jnp
from jax import lax
from jax.experimental import pallas as pl
from jax.experimental.pallas import tpu as pltpu
from jax.experimental.pallas import tpu_sc as plsc

N_NODES = 10000
D = 128
N_EDGES = 320000

NC = 2
NS = 16
L = 16
NW = NC * NS

CHUNK = 128
GC = 8
NG = 10
NCH = GC * NG
EPW = NCH * CHUNK
E_PAD = EPW * NW
N_PAD = 10240
RPT = N_PAD // NS

_mesh = plsc.VectorSubcoreMesh(core_axis_name="c", subcore_axis_name="s")


@functools.partial(
    pl.kernel,
    out_type=jax.ShapeDtypeStruct((NC, N_PAD), jnp.float32),
    mesh=_mesh,
    scratch_types=[
        pltpu.VMEM((GC, CHUNK), jnp.int32),
        pltpu.VMEM((CHUNK,), jnp.float32),
        pltpu.VMEM((RPT,), jnp.float32),
        pltpu.VMEM_SHARED((N_PAD,), jnp.float32),
        pltpu.SemaphoreType.DMA,
    ],
)
def _deg_kernel(dst_hbm, deg_out, idx_v, ones_v, zero_v, deg_sh, sem):
    cid = lax.axis_index("c")
    sid = lax.axis_index("s")
    wid = cid * NS + sid

    zeros = jnp.zeros((L,), jnp.float32)
    ones = jnp.ones((L,), jnp.float32)

    def _zero(i, carry):
        zero_v[pl.ds(i * L, L)] = zeros
        return carry

    lax.fori_loop(0, RPT // L, _zero, 0)

    def _one(i, carry):
        ones_v[pl.ds(i * L, L)] = ones
        return carry

    lax.fori_loop(0, CHUNK // L, _one, 0)

    pltpu.sync_copy(zero_v, deg_sh.at[pl.ds(sid * RPT, RPT)])
    plsc.subcore_barrier()

    def _group(g, carry):
        pltpu.sync_copy(dst_hbm.at[wid, g], idx_v)

        def _scat(k, c2):
            pltpu.async_copy(ones_v, deg_sh.at[idx_v.at[k]], sem, add=True)
            return c2

        lax.fori_loop(0, GC, _scat, 0)

        def _drain(k, c2):
            pltpu.make_async_copy(ones_v, deg_sh.at[idx_v.at[k]], sem).wait()
            return c2

        lax.fori_loop(0, GC, _drain, 0)
        return carry

    lax.fori_loop(0, NG, _group, 0)

    plsc.subcore_barrier()
    pltpu.sync_copy(deg_sh.at[pl.ds(sid * RPT, RPT)],
                    deg_out.at[cid, pl.ds(sid * RPT, RPT)])


@functools.partial(
    pl.kernel,
    out_type=jax.ShapeDtypeStruct((NC, N_PAD, D), jnp.float32),
    mesh=_mesh,
    scratch_types=[
        pltpu.VMEM((GC, CHUNK), jnp.int32),
        pltpu.VMEM((GC, CHUNK), jnp.int32),
        pltpu.VMEM((GC, CHUNK), jnp.int32),
        pltpu.VMEM((GC, CHUNK), jnp.int32),
        pltpu.VMEM((CHUNK, D), jnp.float32),
        pltpu.VMEM((CHUNK, D), jnp.float32),
        pltpu.VMEM_SHARED((N_PAD, D), jnp.float32),
        pltpu.SemaphoreType.DMA,
        pltpu.SemaphoreType.DMA,
        pltpu.SemaphoreType.DMA,
    ],
)
def _agg_kernel(xs_hbm, src_hbm, dst_hbm, part_out,
                idxs0, idxs1, idxd0, idxd1, rows0, rows1, agg_sh,
                sem0, sem1, semi):
    cid = lax.axis_index("c")
    sid = lax.axis_index("s")
    wid = cid * NS + sid

    zeros = jnp.zeros((L,), jnp.float32)

    def _zero(t, carry):
        r = t // (D // L)
        j = t - r * (D // L)
        rows0[r, pl.ds(j * L, L)] = zeros
        return carry

    lax.fori_loop(0, CHUNK * (D // L), _zero, 0)

    for q in range(RPT // CHUNK):
        pltpu.sync_copy(rows0, agg_sh.at[pl.ds(sid * RPT + q * CHUNK, CHUNK)])
    plsc.subcore_barrier()

    pltpu.sync_copy(src_hbm.at[wid, 0], idxs0)
    pltpu.sync_copy(dst_hbm.at[wid, 0], idxd0)
    pltpu.async_copy(src_hbm.at[wid, 1], idxs1, semi)
    pltpu.async_copy(dst_hbm.at[wid, 1], idxd1, semi)
    pltpu.async_copy(xs_hbm.at[idxs0.at[0]], rows0, sem0)

    for g in range(NG):
        a_idxs, a_idxd = (idxs0, idxd0) if g % 2 == 0 else (idxs1, idxd1)
        n_idxs, n_idxd = (idxs1, idxd1) if g % 2 == 0 else (idxs0, idxd0)

        def _step(i, carry, a_idxs=a_idxs, a_idxd=a_idxd):
            k0 = i * 2
            k1 = k0 + 1
            pltpu.async_copy(xs_hbm.at[a_idxs.at[k1]], rows1, sem1)
            pltpu.make_async_copy(xs_hbm.at[a_idxs.at[k0]], rows0, sem0).wait()
            pltpu.sync_copy(rows0, agg_sh.at[a_idxd.at[k0]], add=True)
            pltpu.async_copy(xs_hbm.at[a_idxs.at[k0 + 2]], rows0, sem0)
            pltpu.make_async_copy(xs_hbm.at[a_idxs.at[k1]], rows1, sem1).wait()
            pltpu.sync_copy(rows1, agg_sh.at[a_idxd.at[k1]], add=True)
            return carry

        lax.fori_loop(0, GC // 2 - 1, _step, 0)

        pltpu.async_copy(xs_hbm.at[a_idxs.at[GC - 1]], rows1, sem1)
        pltpu.make_async_copy(xs_hbm.at[a_idxs.at[GC - 2]], rows0, sem0).wait()
        pltpu.sync_copy(rows0, agg_sh.at[a_idxd.at[GC - 2]], add=True)
        if g + 1 < NG:
            pltpu.make_async_copy(src_hbm.at[wid, g + 1], n_idxs, semi).wait()
            pltpu.make_async_copy(dst_hbm.at[wid, g + 1], n_idxd, semi).wait()
            pltpu.async_copy(xs_hbm.at[n_idxs.at[0]], rows0, sem0)
        pltpu.make_async_copy(xs_hbm.at[a_idxs.at[GC - 1]], rows1, sem1).wait()
        pltpu.sync_copy(rows1, agg_sh.at[a_idxd.at[GC - 1]], add=True)
        if g + 2 < NG:
            pltpu.async_copy(src_hbm.at[wid, g + 2], a_idxs, semi)
            pltpu.async_copy(dst_hbm.at[wid, g + 2], a_idxd, semi)

    plsc.subcore_barrier()
    pltpu.sync_copy(agg_sh.at[pl.ds(sid * RPT, RPT)],
                    part_out.at[cid, pl.ds(sid * RPT, RPT)])


def _scale_body(x_ref, degp_ref, xs_ref):
    dis = lax.rsqrt(degp_ref[0] + degp_ref[1] + 1.0)
    xs_ref[...] = x_ref[...] * dis


def _scale_call(x_pad, degp3):
    blk = N_PAD // 8
    return pl.pallas_call(
        _scale_body,
        grid=(8,),
        in_specs=[
            pl.BlockSpec((blk, D), lambda i: (i, 0)),
            pl.BlockSpec((NC, blk, 1), lambda i: (0, i, 0)),
        ],
        out_specs=pl.BlockSpec((blk, D), lambda i: (i, 0)),
        out_shape=jax.ShapeDtypeStruct((N_PAD, D), jnp.float32),
    )(x_pad, degp3)


def _out_body(p_ref, xs_ref, degp_ref, wmu_ref, bmu_ref, wls_ref, bls_ref,
              mu_ref, ls_ref):
    dis = lax.rsqrt(degp_ref[0] + degp_ref[1] + 1.0)
    y = (p_ref[0] + p_ref[1] + xs_ref[...]) * dis
    mu_ref[...] = (
        jnp.dot(y, wmu_ref[...], preferred_element_type=jnp.float32)
        + bmu_ref[...]
    )
    ls_ref[...] = (
        jnp.dot(y, wls_ref[...], preferred_element_type=jnp.float32)
        + bls_ref[...]
    )


def _out_call(part, xs, degp3, W_mu, b_mu, W_logstd, b_logstd):
    blk = 1000
    grid = N_NODES // blk
    wspec = pl.BlockSpec((D, D), lambda i: (0, 0))
    bspec = pl.BlockSpec((1, D), lambda i: (0, 0))
    return pl.pallas_call(
        _out_body,
        grid=(grid,),
        in_specs=[
            pl.BlockSpec((NC, blk, D), lambda i: (0, i, 0)),
            pl.BlockSpec((blk, D), lambda i: (i, 0)),
            pl.BlockSpec((NC, blk, 1), lambda i: (0, i, 0)),
            wspec, bspec, wspec, bspec,
        ],
        out_specs=[
            pl.BlockSpec((blk, D), lambda i: (i, 0)),
            pl.BlockSpec((blk, D), lambda i: (i, 0)),
        ],
        out_shape=[
            jax.ShapeDtypeStruct((N_NODES, D), jnp.float32),
            jax.ShapeDtypeStruct((N_NODES, D), jnp.float32),
        ],
    )(part, xs, degp3, W_mu, b_mu, W_logstd, b_logstd)


def kernel(x, edge_index, W_mu, b_mu, W_logstd, b_logstd):
    src = edge_index[0].astype(jnp.int32)
    dst = edge_index[1].astype(jnp.int32)
    pad = N_NODES + (jnp.arange(E_PAD - N_EDGES, dtype=jnp.int32)
                     % (N_PAD - N_NODES))
    srcb = jnp.concatenate([src, pad]).reshape(NW, NG, GC, CHUNK)
    dstb = jnp.concatenate([dst, pad]).reshape(NW, NG, GC, CHUNK)
    x_pad = jnp.zeros((N_PAD, D), jnp.float32).at[:N_NODES].set(x)

    degp = _deg_kernel(dstb)
    degp3 = degp.reshape(NC, N_PAD, 1)
    xs = _scale_call(x_pad, degp3)
    part = _agg_kernel(xs, srcb, dstb)
    mu, logstd = _out_call(part, xs, degp3, W_mu, b_mu.reshape(1, D),
                           W_logstd, b_logstd.reshape(1, D))
    return (mu, logstd)


# speedup vs baseline: 53.3837x; 1.0060x over previous
import functools

import jax
import jax.numpy as jnp
from jax import lax
from jax.experimental import pallas as pl
from jax.experimental.pallas import tpu as pltpu
from jax.experimental.pallas import tpu_sc as plsc

N_NODES = 10000
D = 128
N_EDGES = 320000

NC = 2
NS = 16
L = 16
NW = NC * NS

CHUNK = 128
GC = 8
NG = 10
NCH = GC * NG
EPW = NCH * CHUNK
E_PAD = EPW * NW
N_PAD = 10240
RPT = N_PAD // NS

_mesh = plsc.VectorSubcoreMesh(core_axis_name="c", subcore_axis_name="s")


@functools.partial(
    pl.kernel,
    out_type=jax.ShapeDtypeStruct((NC, N_PAD), jnp.float32),
    mesh=_mesh,
    scratch_types=[
        pltpu.VMEM((GC, CHUNK), jnp.int32),
        pltpu.VMEM((CHUNK,), jnp.float32),
        pltpu.VMEM((RPT,), jnp.float32),
        pltpu.VMEM_SHARED((N_PAD,), jnp.float32),
        pltpu.SemaphoreType.DMA,
    ],
)
def _deg_kernel(dst_hbm, deg_out, idx_v, ones_v, zero_v, deg_sh, sem):
    cid = lax.axis_index("c")
    sid = lax.axis_index("s")
    wid = cid * NS + sid

    zeros = jnp.zeros((L,), jnp.float32)
    ones = jnp.ones((L,), jnp.float32)

    def _zero(i, carry):
        zero_v[pl.ds(i * L, L)] = zeros
        return carry

    lax.fori_loop(0, RPT // L, _zero, 0)

    def _one(i, carry):
        ones_v[pl.ds(i * L, L)] = ones
        return carry

    lax.fori_loop(0, CHUNK // L, _one, 0)

    pltpu.sync_copy(zero_v, deg_sh.at[pl.ds(sid * RPT, RPT)])
    plsc.subcore_barrier()

    def _group(g, carry):
        pltpu.sync_copy(dst_hbm.at[wid, g], idx_v)

        def _scat(k, c2):
            pltpu.async_copy(ones_v, deg_sh.at[idx_v.at[k]], sem, add=True)
            return c2

        lax.fori_loop(0, GC, _scat, 0)

        def _drain(k, c2):
            pltpu.make_async_copy(ones_v, deg_sh.at[idx_v.at[k]], sem).wait()
            return c2

        lax.fori_loop(0, GC, _drain, 0)
        return carry

    lax.fori_loop(0, NG, _group, 0)

    plsc.subcore_barrier()
    pltpu.sync_copy(deg_sh.at[pl.ds(sid * RPT, RPT)],
                    deg_out.at[cid, pl.ds(sid * RPT, RPT)])


@functools.partial(
    pl.kernel,
    out_type=jax.ShapeDtypeStruct((NC, N_PAD, D), jnp.float32),
    mesh=_mesh,
    scratch_types=[
        pltpu.VMEM((GC, CHUNK), jnp.int32),
        pltpu.VMEM((GC, CHUNK), jnp.int32),
        pltpu.VMEM((GC, CHUNK), jnp.int32),
        pltpu.VMEM((GC, CHUNK), jnp.int32),
        pltpu.VMEM((CHUNK, D), jnp.float32),
        pltpu.VMEM((CHUNK, D), jnp.float32),
        pltpu.VMEM_SHARED((N_PAD, D), jnp.float32),
        pltpu.SemaphoreType.DMA,
        pltpu.SemaphoreType.DMA,
        pltpu.SemaphoreType.DMA,
    ],
)
def _agg_kernel(xs_hbm, src_hbm, dst_hbm, part_out,
                idxs0, idxs1, idxd0, idxd1, rows0, rows1, agg_sh,
                sem0, sem1, semi):
    cid = lax.axis_index("c")
    sid = lax.axis_index("s")
    wid = cid * NS + sid

    zeros = jnp.zeros((L,), jnp.float32)

    def _zero(t, carry):
        r = t // (D // L)
        j = t - r * (D // L)
        rows0[r, pl.ds(j * L, L)] = zeros
        return carry

    lax.fori_loop(0, CHUNK * (D // L), _zero, 0)

    for q in range(RPT // CHUNK):
        pltpu.sync_copy(rows0, agg_sh.at[pl.ds(sid * RPT + q * CHUNK, CHUNK)])
    plsc.subcore_barrier()

    pltpu.sync_copy(src_hbm.at[wid, 0], idxs0)
    pltpu.sync_copy(dst_hbm.at[wid, 0], idxd0)
    pltpu.async_copy(src_hbm.at[wid, 1], idxs1, semi)
    pltpu.async_copy(dst_hbm.at[wid, 1], idxd1, semi)
    pltpu.async_copy(xs_hbm.at[idxs0.at[0]], rows0, sem0)

    for g in range(NG):
        a_idxs, a_idxd = (idxs0, idxd0) if g % 2 == 0 else (idxs1, idxd1)
        n_idxs, n_idxd = (idxs1, idxd1) if g % 2 == 0 else (idxs0, idxd0)

        def _step(i, carry, a_idxs=a_idxs, a_idxd=a_idxd):
            k0 = i * 2
            k1 = k0 + 1
            pltpu.async_copy(xs_hbm.at[a_idxs.at[k1]], rows1, sem1)
            pltpu.make_async_copy(xs_hbm.at[a_idxs.at[k0]], rows0, sem0).wait()
            pltpu.sync_copy(rows0, agg_sh.at[a_idxd.at[k0]], add=True)
            pltpu.async_copy(xs_hbm.at[a_idxs.at[k0 + 2]], rows0, sem0)
            pltpu.make_async_copy(xs_hbm.at[a_idxs.at[k1]], rows1, sem1).wait()
            pltpu.sync_copy(rows1, agg_sh.at[a_idxd.at[k1]], add=True)
            return carry

        lax.fori_loop(0, GC // 2 - 1, _step, 0)

        pltpu.async_copy(xs_hbm.at[a_idxs.at[GC - 1]], rows1, sem1)
        pltpu.make_async_copy(xs_hbm.at[a_idxs.at[GC - 2]], rows0, sem0).wait()
        pltpu.sync_copy(rows0, agg_sh.at[a_idxd.at[GC - 2]], add=True)
        if g + 1 < NG:
            pltpu.make_async_copy(src_hbm.at[wid, g + 1], n_idxs, semi).wait()
            pltpu.make_async_copy(dst_hbm.at[wid, g + 1], n_idxd, semi).wait()
            pltpu.async_copy(xs_hbm.at[n_idxs.at[0]], rows0, sem0)
        pltpu.make_async_copy(xs_hbm.at[a_idxs.at[GC - 1]], rows1, sem1).wait()
        pltpu.sync_copy(rows1, agg_sh.at[a_idxd.at[GC - 1]], add=True)
        if g + 2 < NG:
            pltpu.async_copy(src_hbm.at[wid, g + 2], a_idxs, semi)
            pltpu.async_copy(dst_hbm.at[wid, g + 2], a_idxd, semi)

    plsc.subcore_barrier()
    pltpu.sync_copy(agg_sh.at[pl.ds(sid * RPT, RPT)],
                    part_out.at[cid, pl.ds(sid * RPT, RPT)])


def _scale_body(x_ref, degp_ref, xs_ref):
    blk = xs_ref.shape[0]
    i = pl.program_id(0)
    dis = lax.rsqrt(degp_ref[0] + degp_ref[1] + 1.0)
    rid = i * blk + lax.broadcasted_iota(jnp.int32, (blk, 1), 0)
    xs_ref[...] = jnp.where(rid < N_NODES, x_ref[...] * dis, 0.0)


def _scale_call(x, degp3):
    blk = N_PAD // 8
    return pl.pallas_call(
        _scale_body,
        grid=(8,),
        in_specs=[
            pl.BlockSpec((blk, D), lambda i: (i, 0)),
            pl.BlockSpec((NC, blk, 1), lambda i: (0, i, 0)),
        ],
        out_specs=pl.BlockSpec((blk, D), lambda i: (i, 0)),
        out_shape=jax.ShapeDtypeStruct((N_PAD, D), jnp.float32),
    )(x, degp3)


def _out_body(p_ref, xs_ref, degp_ref, wmu_ref, bmu_ref, wls_ref, bls_ref,
              mu_ref, ls_ref):
    dis = lax.rsqrt(degp_ref[0] + degp_ref[1] + 1.0)
    y = (p_ref[0] + p_ref[1] + xs_ref[...]) * dis
    mu_ref[...] = (
        jnp.dot(y, wmu_ref[...], preferred_element_type=jnp.float32)
        + bmu_ref[...]
    )
    ls_ref[...] = (
        jnp.dot(y, wls_ref[...], preferred_element_type=jnp.float32)
        + bls_ref[...]
    )


def _out_call(part, xs, degp3, W_mu, b_mu, W_logstd, b_logstd):
    blk = 1000
    grid = N_NODES // blk
    wspec = pl.BlockSpec((D, D), lambda i: (0, 0))
    bspec = pl.BlockSpec((1, D), lambda i: (0, 0))
    return pl.pallas_call(
        _out_body,
        grid=(grid,),
        in_specs=[
            pl.BlockSpec((NC, blk, D), lambda i: (0, i, 0)),
            pl.BlockSpec((blk, D), lambda i: (i, 0)),
            pl.BlockSpec((NC, blk, 1), lambda i: (0, i, 0)),
            wspec, bspec, wspec, bspec,
        ],
        out_specs=[
            pl.BlockSpec((blk, D), lambda i: (i, 0)),
            pl.BlockSpec((blk, D), lambda i: (i, 0)),
        ],
        out_shape=[
            jax.ShapeDtypeStruct((N_NODES, D), jnp.float32),
            jax.ShapeDtypeStruct((N_NODES, D), jnp.float32),
        ],
    )(part, xs, degp3, W_mu, b_mu, W_logstd, b_logstd)


def kernel(x, edge_index, W_mu, b_mu, W_logstd, b_logstd):
    src = edge_index[0].astype(jnp.int32)
    dst = edge_index[1].astype(jnp.int32)
    pad = N_NODES + (jnp.arange(E_PAD - N_EDGES, dtype=jnp.int32)
                     % (N_PAD - N_NODES))
    srcb = jnp.concatenate([src, pad]).reshape(NW, NG, GC, CHUNK)
    dstb = jnp.concatenate([dst, pad]).reshape(NW, NG, GC, CHUNK)

    degp = _deg_kernel(dstb)
    degp3 = degp.reshape(NC, N_PAD, 1)
    xs = _scale_call(x, degp3)
    part = _agg_kernel(xs, srcb, dstb)
    mu, logstd = _out_call(part, xs, degp3, W_mu, b_mu.reshape(1, D),
                           W_logstd, b_logstd.reshape(1, D))
    return (mu, logstd)


# speedup vs baseline: 55.0842x; 1.0319x over previous
import functools

import jax
import jax.numpy as jnp
from jax import lax
from jax.experimental import pallas as pl
from jax.experimental.pallas import tpu as pltpu
from jax.experimental.pallas import tpu_sc as plsc

N_NODES = 10000
D = 128
N_EDGES = 320000

NC = 2
NS = 16
L = 16
NW = NC * NS

CHUNK = 128
GC = 8
NG = 10
NCH = GC * NG
EPW = NCH * CHUNK
E_PAD = EPW * NW
N_PAD = 10240
RPT = N_PAD // NS

_mesh = plsc.VectorSubcoreMesh(core_axis_name="c", subcore_axis_name="s")


@functools.partial(
    pl.kernel,
    out_type=jax.ShapeDtypeStruct((NC, N_PAD), jnp.float32),
    mesh=_mesh,
    scratch_types=[
        pltpu.VMEM((NCH, CHUNK), jnp.int32),
        pltpu.VMEM((CHUNK,), jnp.float32),
        pltpu.VMEM((RPT,), jnp.float32),
        pltpu.VMEM_SHARED((N_PAD,), jnp.float32),
        pltpu.SemaphoreType.DMA,
    ],
)
def _deg_kernel(dst_hbm, deg_out, idx_v, ones_v, zero_v, deg_sh, sem):
    cid = lax.axis_index("c")
    sid = lax.axis_index("s")
    wid = cid * NS + sid

    zeros = jnp.zeros((L,), jnp.float32)
    ones = jnp.ones((L,), jnp.float32)

    def _zero(i, carry):
        zero_v[pl.ds(i * L, L)] = zeros
        return carry

    lax.fori_loop(0, RPT // L, _zero, 0)

    def _one(i, carry):
        ones_v[pl.ds(i * L, L)] = ones
        return carry

    lax.fori_loop(0, CHUNK // L, _one, 0)

    pltpu.sync_copy(zero_v, deg_sh.at[pl.ds(sid * RPT, RPT)])
    pltpu.sync_copy(dst_hbm.at[wid], idx_v)
    plsc.subcore_barrier()

    def _scat(k, c2):
        pltpu.async_copy(ones_v, deg_sh.at[idx_v.at[k]], sem, add=True)
        return c2

    lax.fori_loop(0, NCH, _scat, 0)

    def _drain(k, c2):
        pltpu.make_async_copy(ones_v, deg_sh.at[idx_v.at[k]], sem).wait()
        return c2

    lax.fori_loop(0, NCH, _drain, 0)

    plsc.subcore_barrier()
    pltpu.sync_copy(deg_sh.at[pl.ds(sid * RPT, RPT)],
                    deg_out.at[cid, pl.ds(sid * RPT, RPT)])


@functools.partial(
    pl.kernel,
    out_type=jax.ShapeDtypeStruct((NC, N_PAD, D), jnp.float32),
    mesh=_mesh,
    scratch_types=[
        pltpu.VMEM((GC, CHUNK), jnp.int32),
        pltpu.VMEM((GC, CHUNK), jnp.int32),
        pltpu.VMEM((GC, CHUNK), jnp.int32),
        pltpu.VMEM((GC, CHUNK), jnp.int32),
        pltpu.VMEM((CHUNK, D), jnp.float32),
        pltpu.VMEM((CHUNK, D), jnp.float32),
        pltpu.VMEM_SHARED((N_PAD, D), jnp.float32),
        pltpu.SemaphoreType.DMA,
        pltpu.SemaphoreType.DMA,
        pltpu.SemaphoreType.DMA,
    ],
)
def _agg_kernel(xs_hbm, src_hbm, dst_hbm, part_out,
                idxs0, idxs1, idxd0, idxd1, rows0, rows1, agg_sh,
                sem0, sem1, semi):
    cid = lax.axis_index("c")
    sid = lax.axis_index("s")
    wid = cid * NS + sid

    zeros = jnp.zeros((L,), jnp.float32)

    def _zero(t, carry):
        r = t // (D // L)
        j = t - r * (D // L)
        rows0[r, pl.ds(j * L, L)] = zeros
        return carry

    lax.fori_loop(0, CHUNK * (D // L), _zero, 0)

    for q in range(RPT // CHUNK):
        pltpu.sync_copy(rows0, agg_sh.at[pl.ds(sid * RPT + q * CHUNK, CHUNK)])
    plsc.subcore_barrier()

    pltpu.sync_copy(src_hbm.at[wid, 0], idxs0)
    pltpu.sync_copy(dst_hbm.at[wid, 0], idxd0)
    pltpu.async_copy(src_hbm.at[wid, 1], idxs1, semi)
    pltpu.async_copy(dst_hbm.at[wid, 1], idxd1, semi)
    pltpu.async_copy(xs_hbm.at[idxs0.at[0]], rows0, sem0)

    for g in range(NG):
        a_idxs, a_idxd = (idxs0, idxd0) if g % 2 == 0 else (idxs1, idxd1)
        n_idxs, n_idxd = (idxs1, idxd1) if g % 2 == 0 else (idxs0, idxd0)

        def _step(i, carry, a_idxs=a_idxs, a_idxd=a_idxd):
            k0 = i * 2
            k1 = k0 + 1
            pltpu.async_copy(xs_hbm.at[a_idxs.at[k1]], rows1, sem1)
            pltpu.make_async_copy(xs_hbm.at[a_idxs.at[k0]], rows0, sem0).wait()
            pltpu.sync_copy(rows0, agg_sh.at[a_idxd.at[k0]], add=True)
            pltpu.async_copy(xs_hbm.at[a_idxs.at[k0 + 2]], rows0, sem0)
            pltpu.make_async_copy(xs_hbm.at[a_idxs.at[k1]], rows1, sem1).wait()
            pltpu.sync_copy(rows1, agg_sh.at[a_idxd.at[k1]], add=True)
            return carry

        lax.fori_loop(0, GC // 2 - 1, _step, 0)

        pltpu.async_copy(xs_hbm.at[a_idxs.at[GC - 1]], rows1, sem1)
        pltpu.make_async_copy(xs_hbm.at[a_idxs.at[GC - 2]], rows0, sem0).wait()
        pltpu.sync_copy(rows0, agg_sh.at[a_idxd.at[GC - 2]], add=True)
        if g + 1 < NG:
            pltpu.make_async_copy(src_hbm.at[wid, g + 1], n_idxs, semi).wait()
            pltpu.make_async_copy(dst_hbm.at[wid, g + 1], n_idxd, semi).wait()
            pltpu.async_copy(xs_hbm.at[n_idxs.at[0]], rows0, sem0)
        pltpu.make_async_copy(xs_hbm.at[a_idxs.at[GC - 1]], rows1, sem1).wait()
        pltpu.sync_copy(rows1, agg_sh.at[a_idxd.at[GC - 1]], add=True)
        if g + 2 < NG:
            pltpu.async_copy(src_hbm.at[wid, g + 2], a_idxs, semi)
            pltpu.async_copy(dst_hbm.at[wid, g + 2], a_idxd, semi)

    plsc.subcore_barrier()
    pltpu.sync_copy(agg_sh.at[pl.ds(sid * RPT, RPT)],
                    part_out.at[cid, pl.ds(sid * RPT, RPT)])


def _scale_body(x_ref, degp_ref, xs_ref):
    blk = xs_ref.shape[0]
    i = pl.program_id(0)
    dis = lax.rsqrt(degp_ref[0] + degp_ref[1] + 1.0)
    rid = i * blk + lax.broadcasted_iota(jnp.int32, (blk, 1), 0)
    xs_ref[...] = jnp.where(rid < N_NODES, x_ref[...] * dis, 0.0)


def _scale_call(x, degp3):
    blk = N_PAD // 8
    return pl.pallas_call(
        _scale_body,
        grid=(8,),
        in_specs=[
            pl.BlockSpec((blk, D), lambda i: (i, 0)),
            pl.BlockSpec((NC, blk, 1), lambda i: (0, i, 0)),
        ],
        out_specs=pl.BlockSpec((blk, D), lambda i: (i, 0)),
        out_shape=jax.ShapeDtypeStruct((N_PAD, D), jnp.float32),
    )(x, degp3)


def _out_body(p_ref, xs_ref, degp_ref, wmu_ref, bmu_ref, wls_ref, bls_ref,
              mu_ref, ls_ref):
    dis = lax.rsqrt(degp_ref[0] + degp_ref[1] + 1.0)
    y = (p_ref[0] + p_ref[1] + xs_ref[...]) * dis
    mu_ref[...] = (
        jnp.dot(y, wmu_ref[...], preferred_element_type=jnp.float32)
        + bmu_ref[...]
    )
    ls_ref[...] = (
        jnp.dot(y, wls_ref[...], preferred_element_type=jnp.float32)
        + bls_ref[...]
    )


def _out_call(part, xs, degp3, W_mu, b_mu, W_logstd, b_logstd):
    blk = 1000
    grid = N_NODES // blk
    wspec = pl.BlockSpec((D, D), lambda i: (0, 0))
    bspec = pl.BlockSpec((1, D), lambda i: (0, 0))
    return pl.pallas_call(
        _out_body,
        grid=(grid,),
        in_specs=[
            pl.BlockSpec((NC, blk, D), lambda i: (0, i, 0)),
            pl.BlockSpec((blk, D), lambda i: (i, 0)),
            pl.BlockSpec((NC, blk, 1), lambda i: (0, i, 0)),
            wspec, bspec, wspec, bspec,
        ],
        out_specs=[
            pl.BlockSpec((blk, D), lambda i: (i, 0)),
            pl.BlockSpec((blk, D), lambda i: (i, 0)),
        ],
        out_shape=[
            jax.ShapeDtypeStruct((N_NODES, D), jnp.float32),
            jax.ShapeDtypeStruct((N_NODES, D), jnp.float32),
        ],
    )(part, xs, degp3, W_mu, b_mu, W_logstd, b_logstd)


def kernel(x, edge_index, W_mu, b_mu, W_logstd, b_logstd):
    src = edge_index[0].astype(jnp.int32)
    dst = edge_index[1].astype(jnp.int32)
    pad = N_NODES + (jnp.arange(E_PAD - N_EDGES, dtype=jnp.int32)
                     % (N_PAD - N_NODES))
    srcb = jnp.concatenate([src, pad]).reshape(NW, NG, GC, CHUNK)
    dstb = jnp.concatenate([dst, pad]).reshape(NW, NG, GC, CHUNK)

    degp = _deg_kernel(dstb.reshape(NW, NCH, CHUNK))
    degp3 = degp.reshape(NC, N_PAD, 1)
    xs = _scale_call(x, degp3)
    part = _agg_kernel(xs, srcb, dstb)
    mu, logstd = _out_call(part, xs, degp3, W_mu, b_mu.reshape(1, D),
                           W_logstd, b_logstd.reshape(1, D))
    return (mu, logstd)


# speedup vs baseline: 56.2385x; 1.0210x over previous
import functools

import jax
import jax.numpy as jnp
from jax import lax
from jax.experimental import pallas as pl
from jax.experimental.pallas import tpu as pltpu
from jax.experimental.pallas import tpu_sc as plsc

N_NODES = 10000
D = 128
N_EDGES = 320000

NC = 2
NS = 16
L = 16
NW = NC * NS

CHUNK = 128
GC = 8
NG = 10
NCH = GC * NG
EPW = NCH * CHUNK
E_PAD = EPW * NW
N_PAD = 10240
RPT = N_PAD // NS

_mesh = plsc.VectorSubcoreMesh(core_axis_name="c", subcore_axis_name="s")


@functools.partial(
    pl.kernel,
    out_type=jax.ShapeDtypeStruct((NC, N_PAD), jnp.float32),
    mesh=_mesh,
    scratch_types=[
        pltpu.VMEM((NCH, CHUNK), jnp.int32),
        pltpu.VMEM((CHUNK,), jnp.float32),
        pltpu.VMEM((RPT,), jnp.float32),
        pltpu.VMEM_SHARED((N_PAD,), jnp.float32),
        pltpu.SemaphoreType.DMA,
    ],
)
def _deg_kernel(dst_hbm, deg_out, idx_v, ones_v, zero_v, deg_sh, sem):
    cid = lax.axis_index("c")
    sid = lax.axis_index("s")
    wid = cid * NS + sid

    zeros = jnp.zeros((L,), jnp.float32)
    ones = jnp.ones((L,), jnp.float32)

    def _zero(i, carry):
        zero_v[pl.ds(i * L, L)] = zeros
        return carry

    lax.fori_loop(0, RPT // L, _zero, 0)

    def _one(i, carry):
        ones_v[pl.ds(i * L, L)] = ones
        return carry

    lax.fori_loop(0, CHUNK // L, _one, 0)

    pltpu.sync_copy(zero_v, deg_sh.at[pl.ds(sid * RPT, RPT)])
    pltpu.sync_copy(dst_hbm.at[wid], idx_v)
    plsc.subcore_barrier()

    def _scat(k, c2):
        pltpu.async_copy(ones_v, deg_sh.at[idx_v.at[k]], sem, add=True)
        return c2

    lax.fori_loop(0, NCH, _scat, 0)

    def _drain(k, c2):
        pltpu.make_async_copy(ones_v, deg_sh.at[idx_v.at[k]], sem).wait()
        return c2

    lax.fori_loop(0, NCH, _drain, 0)

    plsc.subcore_barrier()
    pltpu.sync_copy(deg_sh.at[pl.ds(sid * RPT, RPT)],
                    deg_out.at[cid, pl.ds(sid * RPT, RPT)])


@functools.partial(
    pl.kernel,
    out_type=jax.ShapeDtypeStruct((NC, N_PAD, D), jnp.float32),
    mesh=_mesh,
    scratch_types=[
        pltpu.VMEM((GC, CHUNK), jnp.int32),
        pltpu.VMEM((GC, CHUNK), jnp.int32),
        pltpu.VMEM((GC, CHUNK), jnp.int32),
        pltpu.VMEM((GC, CHUNK), jnp.int32),
        pltpu.VMEM((CHUNK, D), jnp.float32),
        pltpu.VMEM((CHUNK, D), jnp.float32),
        pltpu.VMEM_SHARED((N_PAD, D), jnp.float32),
        pltpu.SemaphoreType.DMA,
        pltpu.SemaphoreType.DMA,
        pltpu.SemaphoreType.DMA,
    ],
)
def _agg_kernel(xs_hbm, src_hbm, dst_hbm, part_out,
                idxs0, idxs1, idxd0, idxd1, rows0, rows1, agg_sh,
                sem0, sem1, semi):
    cid = lax.axis_index("c")
    sid = lax.axis_index("s")
    wid = cid * NS + sid

    zeros = jnp.zeros((L,), jnp.float32)

    def _zero(t, carry):
        r = t // (D // L)
        j = t - r * (D // L)
        rows0[r, pl.ds(j * L, L)] = zeros
        return carry

    lax.fori_loop(0, CHUNK * (D // L), _zero, 0)

    for q in range(RPT // CHUNK):
        pltpu.sync_copy(rows0, agg_sh.at[pl.ds(sid * RPT + q * CHUNK, CHUNK)])
    plsc.subcore_barrier()

    pltpu.sync_copy(src_hbm.at[wid, 0], idxs0)
    pltpu.sync_copy(dst_hbm.at[wid, 0], idxd0)
    pltpu.async_copy(src_hbm.at[wid, 1], idxs1, semi)
    pltpu.async_copy(dst_hbm.at[wid, 1], idxd1, semi)
    pltpu.async_copy(xs_hbm.at[idxs0.at[0]], rows0, sem0)

    for g in range(NG):
        a_idxs, a_idxd = (idxs0, idxd0) if g % 2 == 0 else (idxs1, idxd1)
        n_idxs, n_idxd = (idxs1, idxd1) if g % 2 == 0 else (idxs0, idxd0)

        def _step(i, carry, a_idxs=a_idxs, a_idxd=a_idxd):
            k0 = i * 2
            k1 = k0 + 1
            pltpu.async_copy(xs_hbm.at[a_idxs.at[k1]], rows1, sem1)
            pltpu.make_async_copy(xs_hbm.at[a_idxs.at[k0]], rows0, sem0).wait()
            pltpu.sync_copy(rows0, agg_sh.at[a_idxd.at[k0]], add=True)
            pltpu.async_copy(xs_hbm.at[a_idxs.at[k0 + 2]], rows0, sem0)
            pltpu.make_async_copy(xs_hbm.at[a_idxs.at[k1]], rows1, sem1).wait()
            pltpu.sync_copy(rows1, agg_sh.at[a_idxd.at[k1]], add=True)
            return carry

        lax.fori_loop(0, GC // 2 - 1, _step, 0)

        pltpu.async_copy(xs_hbm.at[a_idxs.at[GC - 1]], rows1, sem1)
        pltpu.make_async_copy(xs_hbm.at[a_idxs.at[GC - 2]], rows0, sem0).wait()
        pltpu.sync_copy(rows0, agg_sh.at[a_idxd.at[GC - 2]], add=True)
        if g + 1 < NG:
            pltpu.make_async_copy(src_hbm.at[wid, g + 1], n_idxs, semi).wait()
            pltpu.make_async_copy(dst_hbm.at[wid, g + 1], n_idxd, semi).wait()
            pltpu.async_copy(xs_hbm.at[n_idxs.at[0]], rows0, sem0)
        pltpu.make_async_copy(xs_hbm.at[a_idxs.at[GC - 1]], rows1, sem1).wait()
        pltpu.sync_copy(rows1, agg_sh.at[a_idxd.at[GC - 1]], add=True)
        if g + 2 < NG:
            pltpu.async_copy(src_hbm.at[wid, g + 2], a_idxs, semi)
            pltpu.async_copy(dst_hbm.at[wid, g + 2], a_idxd, semi)

    plsc.subcore_barrier()
    pltpu.sync_copy(agg_sh.at[pl.ds(sid * RPT, RPT)],
                    part_out.at[cid, pl.ds(sid * RPT, RPT)])


def _scale_body(x_ref, degp_ref, xs_ref):
    blk = xs_ref.shape[0]
    i = pl.program_id(0)
    dis = lax.rsqrt(degp_ref[0] + degp_ref[1] + 1.0)
    rid = i * blk + lax.broadcasted_iota(jnp.int32, (blk, 1), 0)
    xs_ref[...] = jnp.where(rid < N_NODES, x_ref[...] * dis, 0.0)


def _scale_call(x, degp3):
    blk = N_PAD // 4
    return pl.pallas_call(
        _scale_body,
        grid=(4,),
        in_specs=[
            pl.BlockSpec((blk, D), lambda i: (i, 0)),
            pl.BlockSpec((NC, blk, 1), lambda i: (0, i, 0)),
        ],
        out_specs=pl.BlockSpec((blk, D), lambda i: (i, 0)),
        out_shape=jax.ShapeDtypeStruct((N_PAD, D), jnp.float32),
    )(x, degp3)


def _out_body(p_ref, xs_ref, degp_ref, wmu_ref, bmu_ref, wls_ref, bls_ref,
              mu_ref, ls_ref):
    dis = lax.rsqrt(degp_ref[0] + degp_ref[1] + 1.0)
    y = (p_ref[0] + p_ref[1] + xs_ref[...]) * dis
    mu_ref[...] = (
        jnp.dot(y, wmu_ref[...], preferred_element_type=jnp.float32)
        + bmu_ref[...]
    )
    ls_ref[...] = (
        jnp.dot(y, wls_ref[...], preferred_element_type=jnp.float32)
        + bls_ref[...]
    )


def _out_call(part, xs, degp3, W_mu, b_mu, W_logstd, b_logstd):
    blk = 2000
    grid = N_NODES // blk
    wspec = pl.BlockSpec((D, D), lambda i: (0, 0))
    bspec = pl.BlockSpec((1, D), lambda i: (0, 0))
    return pl.pallas_call(
        _out_body,
        grid=(grid,),
        in_specs=[
            pl.BlockSpec((NC, blk, D), lambda i: (0, i, 0)),
            pl.BlockSpec((blk, D), lambda i: (i, 0)),
            pl.BlockSpec((NC, blk, 1), lambda i: (0, i, 0)),
            wspec, bspec, wspec, bspec,
        ],
        out_specs=[
            pl.BlockSpec((blk, D), lambda i: (i, 0)),
            pl.BlockSpec((blk, D), lambda i: (i, 0)),
        ],
        out_shape=[
            jax.ShapeDtypeStruct((N_NODES, D), jnp.float32),
            jax.ShapeDtypeStruct((N_NODES, D), jnp.float32),
        ],
    )(part, xs, degp3, W_mu, b_mu, W_logstd, b_logstd)


def kernel(x, edge_index, W_mu, b_mu, W_logstd, b_logstd):
    src = edge_index[0].astype(jnp.int32)
    dst = edge_index[1].astype(jnp.int32)
    pad = N_NODES + (jnp.arange(E_PAD - N_EDGES, dtype=jnp.int32)
                     % (N_PAD - N_NODES))
    srcb = jnp.concatenate([src, pad]).reshape(NW, NG, GC, CHUNK)
    dstb = jnp.concatenate([dst, pad]).reshape(NW, NG, GC, CHUNK)

    degp = _deg_kernel(dstb.reshape(NW, NCH, CHUNK))
    degp3 = degp.reshape(NC, N_PAD, 1)
    xs = _scale_call(x, degp3)
    part = _agg_kernel(xs, srcb, dstb)
    mu, logstd = _out_call(part, xs, degp3, W_mu, b_mu.reshape(1, D),
                           W_logstd, b_logstd.reshape(1, D))
    return (mu, logstd)


# speedup vs baseline: 56.9760x; 1.0131x over previous
import functools

import jax
import jax.numpy as jnp
from jax import lax
from jax.experimental import pallas as pl
from jax.experimental.pallas import tpu as pltpu
from jax.experimental.pallas import tpu_sc as plsc

N_NODES = 10000
D = 128
N_EDGES = 320000

NC = 2
NS = 16
L = 16
NW = NC * NS

CHUNK = 128
GC = 8
NG = 10
NCH = GC * NG
EPW = NCH * CHUNK
E_PAD = EPW * NW
N_PAD = 10240
RPT = N_PAD // NS

_mesh = plsc.VectorSubcoreMesh(core_axis_name="c", subcore_axis_name="s")


@functools.partial(
    pl.kernel,
    out_type=jax.ShapeDtypeStruct((NC, N_PAD), jnp.float32),
    mesh=_mesh,
    scratch_types=[
        pltpu.VMEM((NCH, CHUNK), jnp.int32),
        pltpu.VMEM((CHUNK,), jnp.float32),
        pltpu.VMEM((RPT,), jnp.float32),
        pltpu.VMEM_SHARED((N_PAD,), jnp.float32),
        pltpu.SemaphoreType.DMA,
        pltpu.SemaphoreType.DMA,
    ],
)
def _deg_kernel(dst_hbm, deg_out, idx_v, ones_v, zero_v, deg_sh, sem, semi):
    cid = lax.axis_index("c")
    sid = lax.axis_index("s")
    wid = cid * NS + sid

    pltpu.async_copy(dst_hbm.at[wid], idx_v, semi)

    zeros = jnp.zeros((L,), jnp.float32)
    ones = jnp.ones((L,), jnp.float32)

    def _zero(i, carry):
        zero_v[pl.ds(i * L, L)] = zeros
        return carry

    lax.fori_loop(0, RPT // L, _zero, 0)

    def _one(i, carry):
        ones_v[pl.ds(i * L, L)] = ones
        return carry

    lax.fori_loop(0, CHUNK // L, _one, 0)

    pltpu.sync_copy(zero_v, deg_sh.at[pl.ds(sid * RPT, RPT)])
    pltpu.make_async_copy(dst_hbm.at[wid], idx_v, semi).wait()
    plsc.subcore_barrier()

    def _scat(k, c2):
        pltpu.async_copy(ones_v, deg_sh.at[idx_v.at[k]], sem, add=True)
        return c2

    lax.fori_loop(0, NCH, _scat, 0)

    def _drain(k, c2):
        pltpu.make_async_copy(ones_v, deg_sh.at[idx_v.at[k]], sem).wait()
        return c2

    lax.fori_loop(0, NCH, _drain, 0)

    plsc.subcore_barrier()
    pltpu.sync_copy(deg_sh.at[pl.ds(sid * RPT, RPT)],
                    deg_out.at[cid, pl.ds(sid * RPT, RPT)])


@functools.partial(
    pl.kernel,
    out_type=jax.ShapeDtypeStruct((NC, N_PAD, D), jnp.float32),
    mesh=_mesh,
    scratch_types=[
        pltpu.VMEM((GC, CHUNK), jnp.int32),
        pltpu.VMEM((GC, CHUNK), jnp.int32),
        pltpu.VMEM((GC, CHUNK), jnp.int32),
        pltpu.VMEM((GC, CHUNK), jnp.int32),
        pltpu.VMEM((CHUNK, D), jnp.float32),
        pltpu.VMEM((CHUNK, D), jnp.float32),
        pltpu.VMEM_SHARED((N_PAD, D), jnp.float32),
        pltpu.SemaphoreType.DMA,
        pltpu.SemaphoreType.DMA,
        pltpu.SemaphoreType.DMA,
        pltpu.SemaphoreType.DMA,
    ],
)
def _agg_kernel(xs_hbm, src_hbm, dst_hbm, part_out,
                idxs0, idxs1, idxd0, idxd1, rows0, rows1, agg_sh,
                sem0, sem1, semi, semi2):
    cid = lax.axis_index("c")
    sid = lax.axis_index("s")
    wid = cid * NS + sid

    pltpu.async_copy(src_hbm.at[wid, 0], idxs0, semi2)
    pltpu.async_copy(dst_hbm.at[wid, 0], idxd0, semi2)
    pltpu.async_copy(src_hbm.at[wid, 1], idxs1, semi)
    pltpu.async_copy(dst_hbm.at[wid, 1], idxd1, semi)

    zeros = jnp.zeros((L,), jnp.float32)

    def _zero(t, carry):
        r = t // (D // L)
        j = t - r * (D // L)
        rows0[r, pl.ds(j * L, L)] = zeros
        return carry

    lax.fori_loop(0, CHUNK * (D // L), _zero, 0)

    for q in range(RPT // CHUNK):
        pltpu.sync_copy(rows0, agg_sh.at[pl.ds(sid * RPT + q * CHUNK, CHUNK)])

    pltpu.make_async_copy(src_hbm.at[wid, 0], idxs0, semi2).wait()
    pltpu.make_async_copy(dst_hbm.at[wid, 0], idxd0, semi2).wait()
    pltpu.async_copy(xs_hbm.at[idxs0.at[0]], rows0, sem0)
    plsc.subcore_barrier()

    for g in range(NG):
        a_idxs, a_idxd = (idxs0, idxd0) if g % 2 == 0 else (idxs1, idxd1)
        n_idxs, n_idxd = (idxs1, idxd1) if g % 2 == 0 else (idxs0, idxd0)

        def _step(i, carry, a_idxs=a_idxs, a_idxd=a_idxd):
            k0 = i * 2
            k1 = k0 + 1
            pltpu.async_copy(xs_hbm.at[a_idxs.at[k1]], rows1, sem1)
            pltpu.make_async_copy(xs_hbm.at[a_idxs.at[k0]], rows0, sem0).wait()
            pltpu.sync_copy(rows0, agg_sh.at[a_idxd.at[k0]], add=True)
            pltpu.async_copy(xs_hbm.at[a_idxs.at[k0 + 2]], rows0, sem0)
            pltpu.make_async_copy(xs_hbm.at[a_idxs.at[k1]], rows1, sem1).wait()
            pltpu.sync_copy(rows1, agg_sh.at[a_idxd.at[k1]], add=True)
            return carry

        lax.fori_loop(0, GC // 2 - 1, _step, 0)

        pltpu.async_copy(xs_hbm.at[a_idxs.at[GC - 1]], rows1, sem1)
        pltpu.make_async_copy(xs_hbm.at[a_idxs.at[GC - 2]], rows0, sem0).wait()
        pltpu.sync_copy(rows0, agg_sh.at[a_idxd.at[GC - 2]], add=True)
        if g + 1 < NG:
            pltpu.make_async_copy(src_hbm.at[wid, g + 1], n_idxs, semi).wait()
            pltpu.make_async_copy(dst_hbm.at[wid, g + 1], n_idxd, semi).wait()
            pltpu.async_copy(xs_hbm.at[n_idxs.at[0]], rows0, sem0)
        pltpu.make_async_copy(xs_hbm.at[a_idxs.at[GC - 1]], rows1, sem1).wait()
        pltpu.sync_copy(rows1, agg_sh.at[a_idxd.at[GC - 1]], add=True)
        if g + 2 < NG:
            pltpu.async_copy(src_hbm.at[wid, g + 2], a_idxs, semi)
            pltpu.async_copy(dst_hbm.at[wid, g + 2], a_idxd, semi)

    plsc.subcore_barrier()
    pltpu.sync_copy(agg_sh.at[pl.ds(sid * RPT, RPT)],
                    part_out.at[cid, pl.ds(sid * RPT, RPT)])


def _scale_body(x_ref, degp_ref, xs_ref):
    blk = xs_ref.shape[0]
    i = pl.program_id(0)
    dis = lax.rsqrt(degp_ref[0] + degp_ref[1] + 1.0)
    rid = i * blk + lax.broadcasted_iota(jnp.int32, (blk, 1), 0)
    xs_ref[...] = jnp.where(rid < N_NODES, x_ref[...] * dis, 0.0)


def _scale_call(x, degp3):
    blk = N_PAD // 4
    return pl.pallas_call(
        _scale_body,
        grid=(4,),
        in_specs=[
            pl.BlockSpec((blk, D), lambda i: (i, 0)),
            pl.BlockSpec((NC, blk, 1), lambda i: (0, i, 0)),
        ],
        out_specs=pl.BlockSpec((blk, D), lambda i: (i, 0)),
        out_shape=jax.ShapeDtypeStruct((N_PAD, D), jnp.float32),
    )(x, degp3)


def _out_body(p_ref, xs_ref, degp_ref, wmu_ref, bmu_ref, wls_ref, bls_ref,
              mu_ref, ls_ref):
    dis = lax.rsqrt(degp_ref[0] + degp_ref[1] + 1.0)
    y = (p_ref[0] + p_ref[1] + xs_ref[...]) * dis
    mu_ref[...] = (
        jnp.dot(y, wmu_ref[...], preferred_element_type=jnp.float32)
        + bmu_ref[...]
    )
    ls_ref[...] = (
        jnp.dot(y, wls_ref[...], preferred_element_type=jnp.float32)
        + bls_ref[...]
    )


def _out_call(part, xs, degp3, W_mu, b_mu, W_logstd, b_logstd):
    blk = 2000
    grid = N_NODES // blk
    wspec = pl.BlockSpec((D, D), lambda i: (0, 0))
    bspec = pl.BlockSpec((1, D), lambda i: (0, 0))
    return pl.pallas_call(
        _out_body,
        grid=(grid,),
        in_specs=[
            pl.BlockSpec((NC, blk, D), lambda i: (0, i, 0)),
            pl.BlockSpec((blk, D), lambda i: (i, 0)),
            pl.BlockSpec((NC, blk, 1), lambda i: (0, i, 0)),
            wspec, bspec, wspec, bspec,
        ],
        out_specs=[
            pl.BlockSpec((blk, D), lambda i: (i, 0)),
            pl.BlockSpec((blk, D), lambda i: (i, 0)),
        ],
        out_shape=[
            jax.ShapeDtypeStruct((N_NODES, D), jnp.float32),
            jax.ShapeDtypeStruct((N_NODES, D), jnp.float32),
        ],
    )(part, xs, degp3, W_mu, b_mu, W_logstd, b_logstd)


def kernel(x, edge_index, W_mu, b_mu, W_logstd, b_logstd):
    src = edge_index[0].astype(jnp.int32)
    dst = edge_index[1].astype(jnp.int32)
    pad = N_NODES + (jnp.arange(E_PAD - N_EDGES, dtype=jnp.int32)
                     % (N_PAD - N_NODES))
    srcb = jnp.concatenate([src, pad]).reshape(NW, NG, GC, CHUNK)
    dstb = jnp.concatenate([dst, pad]).reshape(NW, NG, GC, CHUNK)

    degp = _deg_kernel(dstb.reshape(NW, NCH, CHUNK))
    degp3 = degp.reshape(NC, N_PAD, 1)
    xs = _scale_call(x, degp3)
    part = _agg_kernel(xs, srcb, dstb)
    mu, logstd = _out_call(part, xs, degp3, W_mu, b_mu.reshape(1, D),
                           W_logstd, b_logstd.reshape(1, D))
    return (mu, logstd)
